# baseline (speedup 1.0000x reference)
import functools

import jax
import jax.numpy as jnp
from jax import lax
from jax.experimental import pallas as pl
from jax.experimental.pallas import tpu as pltpu
from jax.experimental.pallas import tpu_sc as plsc

_GI = 128
_NBUF = 4


def _make_sc_gather_pe(n_rows, hidden, seq_len, chunk, n_workers, nc):
    rows_per_w = n_rows // n_workers
    n_chunks = rows_per_w // chunk
    assert n_chunks % _NBUF == 0
    assert chunk % seq_len == 0
    reps = chunk // seq_len
    splits = []
    s = 0
    while s < chunk:
        splits.append((s, min(_GI, chunk - s)))
        s += _GI

    mesh = plsc.VectorSubcoreMesh(core_axis_name="c", subcore_axis_name="s")

    @functools.partial(
        pl.kernel,
        out_type=jax.ShapeDtypeStruct((n_rows, hidden), jnp.float32),
        mesh=mesh,
        compiler_params=pltpu.CompilerParams(use_tc_tiling_on_sc=False),
        scratch_types=[
            pltpu.VMEM((seq_len, hidden), jnp.float32),
            tuple(pltpu.VMEM((chunk,), jnp.int32) for _ in range(_NBUF)),
            tuple(pltpu.VMEM((chunk, hidden), jnp.float32) for _ in range(_NBUF)),
            tuple(pltpu.SemaphoreType.DMA for _ in range(_NBUF)),
            tuple(pltpu.SemaphoreType.DMA for _ in range(_NBUF)),
            tuple(pltpu.SemaphoreType.DMA for _ in range(_NBUF)),
        ],
    )
    def gather_pe(table, idxf, pe, out, pe_v, idxs, rows,
                  isems, gsems, ssems):
        wid = lax.axis_index("s") * nc + lax.axis_index("c")
        base = wid * rows_per_w

        pltpu.sync_copy(pe, pe_v)

        def idx_copy(g, b):
            pltpu.async_copy(idxf.at[pl.ds(base + g * chunk, chunk)],
                             idxs[b], isems[b])

        def idx_wait(b):
            pltpu.make_async_copy(idxf.at[pl.ds(0, chunk)],
                                  idxs[b], isems[b]).wait()

        def prefill(b):
            @plsc.parallel_loop(0, seq_len, unroll=8)
            def _(p):
                for c in range(hidden // 16):
                    sl = pl.ds(c * 16, 16)
                    v = pe_v[p, sl]
                    for r in range(reps):
                        rows[b][p + r * seq_len, sl] = v

        def gather_add(b):
            pltpu.async_copy(table.at[idxs[b]], rows[b], gsems[b], add=True)

        def gather_wait(b):
            pltpu.make_async_copy(table.at[pl.ds(0, chunk)],
                                  rows[b], gsems[b]).wait()

        def store(g, b):
            pltpu.async_copy(rows[b],
                             out.at[pl.ds(base + g * chunk, chunk)], ssems[b])

        def store_wait(b):
            pltpu.make_async_copy(rows[b],
                                  out.at[pl.ds(0, chunk)], ssems[b]).wait()

        def prep(g, b, drain_store):
            if drain_store:
                @pl.when(g >= _NBUF)
                def _():
                    store_wait(b)
            prefill(b)
            idx_wait(b)
            gather_add(b)

        for b in range(_NBUF):
            idx_copy(b, b)
        for g in range(_NBUF - 1):
            prep(g, g, drain_store=False)

        def ring_step(i, _):
            for b in range(_NBUF):
                g = _NBUF * i + b
                gp = g + _NBUF - 1

                pb = (b + _NBUF - 1) % _NBUF

                @pl.when(gp < n_chunks)
                def _():
                    prep(gp, pb, drain_store=True)

                gather_wait(b)

                @pl.when(g + _NBUF < n_chunks)
                def _():
                    idx_copy(g + _NBUF, b)

                store(g, b)
            return 0

        lax.fori_loop(0, n_chunks // _NBUF, ring_step, 0)
        for b in range(_NBUF):
            store_wait(b)

    return gather_pe


def kernel(x, table, pe):
    batch, seq = x.shape
    _, hidden = table.shape
    n_rows = batch * seq

    n_workers = 32
    nc = 2
    chunk = 2 * seq

    idxf = x.astype(jnp.int32).reshape(n_rows)
    pe2 = pe[0, :seq, :]

    fn = _make_sc_gather_pe(n_rows, hidden, seq, chunk, n_workers, nc)
    out = fn(table, idxf, pe2)
    return out.reshape(batch, seq, hidden)


# speedup vs baseline: 1.0010x; 1.0010x over previous
import functools

import jax
import jax.numpy as jnp
from jax import lax
from jax.experimental import pallas as pl
from jax.experimental.pallas import tpu as pltpu
from jax.experimental.pallas import tpu_sc as plsc

_NBUF = 4


def _make_sc_gather_pe(batch, seq, hidden, n_workers, nc):
    rows_per_w = batch // n_workers
    n_chunks = rows_per_w
    assert n_chunks % _NBUF == 0

    mesh = plsc.VectorSubcoreMesh(core_axis_name="c", subcore_axis_name="s")

    @functools.partial(
        pl.kernel,
        out_type=jax.ShapeDtypeStruct((batch, seq, hidden), jnp.float32),
        mesh=mesh,
        compiler_params=pltpu.CompilerParams(use_tc_tiling_on_sc=False),
        scratch_types=[
            pltpu.VMEM((seq, hidden), jnp.float32),
            tuple(pltpu.VMEM((seq,), jnp.int32) for _ in range(_NBUF)),
            tuple(pltpu.VMEM((seq, hidden), jnp.float32) for _ in range(_NBUF)),
            tuple(pltpu.SemaphoreType.DMA for _ in range(_NBUF)),
            tuple(pltpu.SemaphoreType.DMA for _ in range(_NBUF)),
            tuple(pltpu.SemaphoreType.DMA for _ in range(_NBUF)),
        ],
    )
    def gather_pe(table, x, pe, out, pe_v, idxs, rows, isems, gsems, ssems):
        wid = lax.axis_index("s") * nc + lax.axis_index("c")
        base = wid * rows_per_w

        pltpu.sync_copy(pe, pe_v)

        def idx_copy(g, b):
            pltpu.async_copy(x.at[base + g], idxs[b], isems[b])

        def idx_wait(b):
            pltpu.make_async_copy(x.at[0], idxs[b], isems[b]).wait()

        def prefill(b):
            @plsc.parallel_loop(0, seq, unroll=8)
            def _(p):
                for c in range(hidden // 16):
                    sl = pl.ds(c * 16, 16)
                    rows[b][p, sl] = pe_v[p, sl]

        def gather_add(b):
            pltpu.async_copy(table.at[idxs[b]], rows[b], gsems[b], add=True)

        def gather_wait(b):
            pltpu.make_async_copy(table.at[pl.ds(0, seq)],
                                  rows[b], gsems[b]).wait()

        def store(g, b):
            pltpu.async_copy(rows[b], out.at[base + g], ssems[b])

        def store_wait(b):
            pltpu.make_async_copy(rows[b], out.at[0], ssems[b]).wait()

        def prep(g, b, drain_store):
            if drain_store:
                @pl.when(g >= _NBUF)
                def _():
                    store_wait(b)
            prefill(b)
            idx_wait(b)
            gather_add(b)

        for b in range(_NBUF):
            idx_copy(b, b)
        for g in range(_NBUF - 1):
            prep(g, g, drain_store=False)

        def ring_step(i, _):
            for b in range(_NBUF):
                g = _NBUF * i + b
                gp = g + _NBUF - 1

                pb = (b + _NBUF - 1) % _NBUF

                @pl.when(gp < n_chunks)
                def _():
                    prep(gp, pb, drain_store=True)

                gather_wait(b)

                @pl.when(g + _NBUF < n_chunks)
                def _():
                    idx_copy(g + _NBUF, b)

                store(g, b)
            return 0

        lax.fori_loop(0, n_chunks // _NBUF, ring_step, 0)
        for b in range(_NBUF):
            store_wait(b)

    return gather_pe


def kernel(x, table, pe):
    batch, seq = x.shape
    _, hidden = table.shape

    n_workers = 32
    nc = 2

    fn = _make_sc_gather_pe(batch, seq, hidden, n_workers, nc)
    return fn(table, x.astype(jnp.int32), pe[0, :seq, :])
